# Initial kernel scaffold; baseline (speedup 1.0000x reference)
import jax
import jax.numpy as jnp
from jax.experimental import pallas as pl


def kernel(normal_features, extreme_features, edge_index, batch, W_l, b_l, W_r, W_fc1, b_fc1, W_fc, b_fc):
    raise NotImplementedError("write your pallas kernel here")


# speedup vs baseline: 6.0150x; 6.0150x over previous
import functools

import jax
import jax.numpy as jnp
from jax import lax
from jax.experimental import pallas as pl
from jax.experimental.pallas import tpu as pltpu
from jax.experimental.pallas import tpu_sc as plsc

N_NODES = 10000
N_EDGES = 640000
IN_DIM = 64
D = 2 * IN_DIM
HID = 128
N_GRAPHS = 16

NC = 2
NS = 16
L = 16

E_PER_SUB = N_EDGES // (NC * NS)
CHUNK = 80
N_CHUNKS = E_PER_SUB // CHUNK
ROWS_PER_SUB = 624
ROWS_TAIL = N_NODES - ROWS_PER_SUB * NS
TAIL_BASE = ROWS_PER_SUB * NS

_MESH = plsc.VectorSubcoreMesh(core_axis_name="c", subcore_axis_name="s")


def _sc_feature_sums(x, src, dst, zrows):
    @functools.partial(
        pl.kernel,
        out_type=jax.ShapeDtypeStruct((NC, N_NODES, D), jnp.float32),
        mesh=_MESH,
        scratch_types=[
            pltpu.VMEM((CHUNK,), jnp.int32),
            pltpu.VMEM((CHUNK,), jnp.int32),
            pltpu.VMEM((CHUNK, D), jnp.float32),
            pltpu.VMEM_SHARED((N_NODES, D), jnp.float32),
            pltpu.SemaphoreType.DMA,
        ],
    )
    def k(x_hbm, src_hbm, dst_hbm, zr_hbm, acc_out,
          src_v, dst_v, rows_v, acc_sh, sem):
        cid = lax.axis_index("c")
        sid = lax.axis_index("s")
        base = sid * ROWS_PER_SUB

        pltpu.sync_copy(zr_hbm.at[pl.ds(base, ROWS_PER_SUB)],
                        acc_sh.at[pl.ds(base, ROWS_PER_SUB)])

        @pl.when(sid == NS - 1)
        def _():
            pltpu.sync_copy(zr_hbm.at[pl.ds(TAIL_BASE, ROWS_TAIL)],
                            acc_sh.at[pl.ds(TAIL_BASE, ROWS_TAIL)])

        plsc.subcore_barrier()

        ebase = (cid * NS + sid) * E_PER_SUB

        @pl.loop(0, N_CHUNKS)
        def _(i):
            off = ebase + i * CHUNK
            pltpu.sync_copy(src_hbm.at[pl.ds(off, CHUNK)], src_v)
            pltpu.sync_copy(dst_hbm.at[pl.ds(off, CHUNK)], dst_v)
            pltpu.async_copy(x_hbm.at[src_v], rows_v, sem).wait()
            pltpu.sync_copy(rows_v, acc_sh.at[dst_v], add=True)

        plsc.subcore_barrier()
        pltpu.sync_copy(acc_sh.at[pl.ds(base, ROWS_PER_SUB)],
                        acc_out.at[cid, pl.ds(base, ROWS_PER_SUB)])

        @pl.when(sid == NS - 1)
        def _():
            pltpu.sync_copy(acc_sh.at[pl.ds(TAIL_BASE, ROWS_TAIL)],
                            acc_out.at[cid, pl.ds(TAIL_BASE, ROWS_TAIL)])

    return k(x, src, dst, zrows)


def _sc_degrees(dst, ones_chunk, zdeg):
    @functools.partial(
        pl.kernel,
        out_type=jax.ShapeDtypeStruct((NC, N_NODES, D), jnp.float32),
        mesh=_MESH,
        scratch_types=[
            pltpu.VMEM((CHUNK,), jnp.int32),
            pltpu.VMEM((CHUNK, D), jnp.float32),
            pltpu.VMEM_SHARED((N_NODES, D), jnp.float32),
        ],
    )
    def k(dst_hbm, ones_hbm, zd_hbm, deg_out, dst_v, ones_v, deg_sh):
        cid = lax.axis_index("c")
        sid = lax.axis_index("s")
        base = sid * ROWS_PER_SUB

        pltpu.sync_copy(zd_hbm.at[pl.ds(base, ROWS_PER_SUB)],
                        deg_sh.at[pl.ds(base, ROWS_PER_SUB)])

        @pl.when(sid == NS - 1)
        def _():
            pltpu.sync_copy(zd_hbm.at[pl.ds(TAIL_BASE, ROWS_TAIL)],
                            deg_sh.at[pl.ds(TAIL_BASE, ROWS_TAIL)])

        pltpu.sync_copy(ones_hbm, ones_v)
        plsc.subcore_barrier()

        ebase = (cid * NS + sid) * E_PER_SUB

        @pl.loop(0, N_CHUNKS)
        def _(i):
            off = ebase + i * CHUNK
            pltpu.sync_copy(dst_hbm.at[pl.ds(off, CHUNK)], dst_v)
            pltpu.sync_copy(ones_v, deg_sh.at[dst_v], add=True)

        plsc.subcore_barrier()
        pltpu.sync_copy(deg_sh.at[pl.ds(base, ROWS_PER_SUB)],
                        deg_out.at[cid, pl.ds(base, ROWS_PER_SUB)])

        @pl.when(sid == NS - 1)
        def _():
            pltpu.sync_copy(deg_sh.at[pl.ds(TAIL_BASE, ROWS_TAIL)],
                            deg_out.at[cid, pl.ds(TAIL_BASE, ROWS_TAIL)])

    return k(dst, ones_chunk, zdeg)


def _tc_body(x_ref, acc_ref, deg_ref, batch_ref,
             wl_ref, bl_ref, wr_ref, wf1_ref, bf1_ref, wf_ref, bf_ref,
             out_ref):
    x = x_ref[...]
    acc = acc_ref[0] + acc_ref[1]
    deg = deg_ref[0, :, 0:1] + deg_ref[1, :, 0:1]
    agg_mean = acc / jnp.maximum(deg, 1.0)
    x_gnn = jnp.maximum(
        jnp.dot(agg_mean, wl_ref[...], preferred_element_type=jnp.float32)
        + bl_ref[...]
        + jnp.dot(x, wr_ref[...], preferred_element_type=jnp.float32),
        0.0)
    x_mlp = jnp.maximum(
        jnp.dot(x, wf1_ref[...], preferred_element_type=jnp.float32)
        + bf1_ref[...],
        0.0)
    x_comb = x_gnn + x_mlp

    b = batch_ref[...]
    gids = lax.broadcasted_iota(jnp.int32, (1, N_GRAPHS), 1)
    onehot = (b == gids).astype(jnp.float32)
    g_sum = lax.dot_general(onehot, x_comb,
                            (((0,), (0,)), ((), ())),
                            preferred_element_type=jnp.float32)
    g_cnt = jnp.sum(onehot, axis=0)[:, None]
    gf = g_sum / jnp.maximum(g_cnt, 1.0)
    logits = jnp.dot(gf, wf_ref[...],
                     preferred_element_type=jnp.float32) + bf_ref[...]
    out_ref[...] = jax.nn.sigmoid(logits)


def kernel(normal_features, extreme_features, edge_index, batch,
           W_l, b_l, W_r, W_fc1, b_fc1, W_fc, b_fc):
    x = jnp.concatenate([normal_features, extreme_features], axis=1)
    src = edge_index[0].astype(jnp.int32)
    dst = edge_index[1].astype(jnp.int32)
    zrows = jnp.zeros((N_NODES, D), jnp.float32)
    ones_chunk = jnp.ones((CHUNK, D), jnp.float32)

    acc2 = _sc_feature_sums(x, src, dst, zrows)
    deg2 = _sc_degrees(dst, ones_chunk, zrows)

    out = pl.pallas_call(
        _tc_body,
        out_shape=jax.ShapeDtypeStruct((N_GRAPHS, 1), jnp.float32),
    )(x, acc2, deg2, batch.astype(jnp.int32).reshape(N_NODES, 1),
      W_l, b_l.reshape(1, HID), W_r, W_fc1, b_fc1.reshape(1, HID),
      W_fc, b_fc.reshape(1, 1))
    return out


# speedup vs baseline: 11.4206x; 1.8987x over previous
import functools

import jax
import jax.numpy as jnp
from jax import lax
from jax.experimental import pallas as pl
from jax.experimental.pallas import tpu as pltpu
from jax.experimental.pallas import tpu_sc as plsc

N_NODES = 10000
N_EDGES = 640000
IN_DIM = 64
D = 2 * IN_DIM
HID = 128
N_GRAPHS = 16

NC = 2
NS = 16
NW = NC * NS

E_PER_SUB = N_EDGES // NW
CHUNK = 128
N_MAIN = E_PER_SUB // CHUNK
TAIL = E_PER_SUB - N_MAIN * CHUNK
UNROLL = 4

ROWS_PER_SUB = 624
ROWS_TAIL = N_NODES - ROWS_PER_SUB * NS
TAIL_BASE = ROWS_PER_SUB * NS

_MESH = plsc.VectorSubcoreMesh(core_axis_name="c", subcore_axis_name="s")


def _sc_feature_sums(x, idx_main, idx_tail, zrows):
    @functools.partial(
        pl.kernel,
        out_type=jax.ShapeDtypeStruct((NC, N_NODES, D), jnp.float32),
        mesh=_MESH,
        scratch_types=[
            pltpu.VMEM((2, CHUNK), jnp.int32),
            pltpu.VMEM((2, CHUNK), jnp.int32),
            pltpu.VMEM((2, CHUNK), jnp.int32),
            pltpu.VMEM((2, CHUNK), jnp.int32),
            pltpu.VMEM((CHUNK, D), jnp.float32),
            pltpu.VMEM((CHUNK, D), jnp.float32),
            pltpu.VMEM((2, TAIL), jnp.int32),
            pltpu.VMEM((TAIL, D), jnp.float32),
            pltpu.VMEM_SHARED((N_NODES, D), jnp.float32),
            pltpu.SemaphoreType.DMA,
            pltpu.SemaphoreType.DMA,
            pltpu.SemaphoreType.DMA,
            pltpu.SemaphoreType.DMA,
            pltpu.SemaphoreType.DMA,
            pltpu.SemaphoreType.DMA,
            pltpu.SemaphoreType.DMA,
            pltpu.SemaphoreType.DMA,
        ],
    )
    def k(x_hbm, im_hbm, it_hbm, zr_hbm, acc_out,
          ibuf0, ibuf1, ibuf2, ibuf3, rows0, rows1, tbuf, trows, acc_sh,
          si0, si1, si2, si3, sg0, sg1, ss0, ss1):
        cid = lax.axis_index("c")
        sid = lax.axis_index("s")
        wid = cid * NS + sid
        base = sid * ROWS_PER_SUB
        ibuf = (ibuf0, ibuf1, ibuf2, ibuf3)
        rows = (rows0, rows1)
        si = (si0, si1, si2, si3)
        sg = (sg0, sg1)
        ss = (ss0, ss1)

        pltpu.sync_copy(zr_hbm.at[pl.ds(base, ROWS_PER_SUB)],
                        acc_sh.at[pl.ds(base, ROWS_PER_SUB)])

        @pl.when(sid == NS - 1)
        def _():
            pltpu.sync_copy(zr_hbm.at[pl.ds(TAIL_BASE, ROWS_TAIL)],
                            acc_sh.at[pl.ds(TAIL_BASE, ROWS_TAIL)])

        plsc.subcore_barrier()

        def idx_start(k_, t):
            pltpu.async_copy(im_hbm.at[wid, k_], ibuf[t], si[t])

        def idx_wait(t):
            pltpu.make_async_copy(im_hbm.at[wid, 0], ibuf[t], si[t]).wait()

        def gather_start(t, r):
            pltpu.async_copy(x_hbm.at[ibuf[t].at[0]], rows[r], sg[r])

        def gather_wait(t, r):
            pltpu.make_async_copy(x_hbm.at[ibuf[t].at[0]], rows[r],
                                  sg[r]).wait()

        def scatter_start(t, r):
            pltpu.async_copy(rows[r], acc_sh.at[ibuf[t].at[1]], ss[r],
                             add=True)

        def scatter_wait(t, r):
            pltpu.make_async_copy(rows[r], acc_sh.at[ibuf[t].at[1]],
                                  ss[r]).wait()

        idx_start(0, 0)
        idx_start(1, 1)

        @pl.loop(0, N_MAIN // UNROLL)
        def _(jj):
            for t in range(UNROLL):
                m = jj * UNROLL + t
                r = t % 2

                @pl.when(m >= 2)
                def _():
                    scatter_wait((t + 2) % 4, r)

                @pl.when(m + 2 < N_MAIN)
                def _():
                    idx_start(m + 2, (t + 2) % 4)

                idx_wait(t)
                gather_start(t, r)
                gather_wait(t, r)
                scatter_start(t, r)

        scatter_wait(2, 0)
        scatter_wait(3, 1)
        pltpu.sync_copy(it_hbm.at[wid], tbuf)
        pltpu.async_copy(x_hbm.at[tbuf.at[0]], trows, sg0).wait()
        pltpu.sync_copy(trows, acc_sh.at[tbuf.at[1]], add=True)

        plsc.subcore_barrier()
        pltpu.sync_copy(acc_sh.at[pl.ds(base, ROWS_PER_SUB)],
                        acc_out.at[cid, pl.ds(base, ROWS_PER_SUB)])

        @pl.when(sid == NS - 1)
        def _():
            pltpu.sync_copy(acc_sh.at[pl.ds(TAIL_BASE, ROWS_TAIL)],
                            acc_out.at[cid, pl.ds(TAIL_BASE, ROWS_TAIL)])

    return k(x, idx_main, idx_tail, zrows)


def _sc_degrees(idx_main, idx_tail, ones_chunk, zdeg):
    @functools.partial(
        pl.kernel,
        out_type=jax.ShapeDtypeStruct((NC, N_NODES, D), jnp.float32),
        mesh=_MESH,
        scratch_types=[
            pltpu.VMEM((2, CHUNK), jnp.int32),
            pltpu.VMEM((2, CHUNK), jnp.int32),
            pltpu.VMEM((2, CHUNK), jnp.int32),
            pltpu.VMEM((2, CHUNK), jnp.int32),
            pltpu.VMEM((CHUNK, D), jnp.float32),
            pltpu.VMEM((2, TAIL), jnp.int32),
            pltpu.VMEM_SHARED((N_NODES, D), jnp.float32),
            pltpu.SemaphoreType.DMA,
            pltpu.SemaphoreType.DMA,
            pltpu.SemaphoreType.DMA,
            pltpu.SemaphoreType.DMA,
            pltpu.SemaphoreType.DMA,
            pltpu.SemaphoreType.DMA,
        ],
    )
    def k(im_hbm, it_hbm, ones_hbm, zd_hbm, deg_out,
          ibuf0, ibuf1, ibuf2, ibuf3, ones_v, tbuf, deg_sh,
          si0, si1, si2, si3, ss0, ss1):
        cid = lax.axis_index("c")
        sid = lax.axis_index("s")
        wid = cid * NS + sid
        base = sid * ROWS_PER_SUB
        ibuf = (ibuf0, ibuf1, ibuf2, ibuf3)
        si = (si0, si1, si2, si3)
        ss = (ss0, ss1)

        pltpu.sync_copy(zd_hbm.at[pl.ds(base, ROWS_PER_SUB)],
                        deg_sh.at[pl.ds(base, ROWS_PER_SUB)])

        @pl.when(sid == NS - 1)
        def _():
            pltpu.sync_copy(zd_hbm.at[pl.ds(TAIL_BASE, ROWS_TAIL)],
                            deg_sh.at[pl.ds(TAIL_BASE, ROWS_TAIL)])

        pltpu.sync_copy(ones_hbm, ones_v)
        plsc.subcore_barrier()

        def idx_start(k_, t):
            pltpu.async_copy(im_hbm.at[wid, k_], ibuf[t], si[t])

        def idx_wait(t):
            pltpu.make_async_copy(im_hbm.at[wid, 0], ibuf[t], si[t]).wait()

        def scatter_start(t, r):
            pltpu.async_copy(ones_v, deg_sh.at[ibuf[t].at[1]], ss[r],
                             add=True)

        def scatter_wait(t, r):
            pltpu.make_async_copy(ones_v, deg_sh.at[ibuf[t].at[1]],
                                  ss[r]).wait()

        idx_start(0, 0)
        idx_start(1, 1)

        @pl.loop(0, N_MAIN // UNROLL)
        def _(jj):
            for t in range(UNROLL):
                m = jj * UNROLL + t
                r = t % 2

                @pl.when(m >= 2)
                def _():
                    scatter_wait((t + 2) % 4, r)

                @pl.when(m + 2 < N_MAIN)
                def _():
                    idx_start(m + 2, (t + 2) % 4)

                idx_wait(t)
                scatter_start(t, r)

        scatter_wait(2, 0)
        scatter_wait(3, 1)
        pltpu.sync_copy(it_hbm.at[wid], tbuf)
        pltpu.sync_copy(ones_v.at[pl.ds(0, TAIL)], deg_sh.at[tbuf.at[1]],
                        add=True)

        plsc.subcore_barrier()
        pltpu.sync_copy(deg_sh.at[pl.ds(base, ROWS_PER_SUB)],
                        deg_out.at[cid, pl.ds(base, ROWS_PER_SUB)])

        @pl.when(sid == NS - 1)
        def _():
            pltpu.sync_copy(deg_sh.at[pl.ds(TAIL_BASE, ROWS_TAIL)],
                            deg_out.at[cid, pl.ds(TAIL_BASE, ROWS_TAIL)])

    return k(idx_main, idx_tail, ones_chunk, zdeg)


def _tc_body(x_ref, acc_ref, deg_ref, batch_ref,
             wl_ref, bl_ref, wr_ref, wf1_ref, bf1_ref, wf_ref, bf_ref,
             out_ref):
    x = x_ref[...]
    acc = acc_ref[0] + acc_ref[1]
    deg = deg_ref[0, :, 0:1] + deg_ref[1, :, 0:1]
    agg_mean = acc / jnp.maximum(deg, 1.0)
    x_gnn = jnp.maximum(
        jnp.dot(agg_mean, wl_ref[...], preferred_element_type=jnp.float32)
        + bl_ref[...]
        + jnp.dot(x, wr_ref[...], preferred_element_type=jnp.float32),
        0.0)
    x_mlp = jnp.maximum(
        jnp.dot(x, wf1_ref[...], preferred_element_type=jnp.float32)
        + bf1_ref[...],
        0.0)
    x_comb = x_gnn + x_mlp

    b = batch_ref[...]
    gids = lax.broadcasted_iota(jnp.int32, (1, N_GRAPHS), 1)
    onehot = (b == gids).astype(jnp.float32)
    g_sum = lax.dot_general(onehot, x_comb,
                            (((0,), (0,)), ((), ())),
                            preferred_element_type=jnp.float32)
    g_cnt = jnp.sum(onehot, axis=0)[:, None]
    gf = g_sum / jnp.maximum(g_cnt, 1.0)
    logits = jnp.dot(gf, wf_ref[...],
                     preferred_element_type=jnp.float32) + bf_ref[...]
    out_ref[...] = jax.nn.sigmoid(logits)


def kernel(normal_features, extreme_features, edge_index, batch,
           W_l, b_l, W_r, W_fc1, b_fc1, W_fc, b_fc):
    x = jnp.concatenate([normal_features, extreme_features], axis=1)
    src = edge_index[0].astype(jnp.int32).reshape(NW, E_PER_SUB)
    dst = edge_index[1].astype(jnp.int32).reshape(NW, E_PER_SUB)
    idx_main = jnp.stack(
        [src[:, :N_MAIN * CHUNK].reshape(NW, N_MAIN, CHUNK),
         dst[:, :N_MAIN * CHUNK].reshape(NW, N_MAIN, CHUNK)], axis=2)
    idx_tail = jnp.stack(
        [src[:, N_MAIN * CHUNK:], dst[:, N_MAIN * CHUNK:]], axis=1)
    zrows = jnp.zeros((N_NODES, D), jnp.float32)
    ones_chunk = jnp.ones((CHUNK, D), jnp.float32)

    acc2 = _sc_feature_sums(x, idx_main, idx_tail, zrows)
    deg2 = _sc_degrees(idx_main, idx_tail, ones_chunk, zrows)

    out = pl.pallas_call(
        _tc_body,
        out_shape=jax.ShapeDtypeStruct((N_GRAPHS, 1), jnp.float32),
    )(x, acc2, deg2, batch.astype(jnp.int32).reshape(N_NODES, 1),
      W_l, b_l.reshape(1, HID), W_r, W_fc1, b_fc1.reshape(1, HID),
      W_fc, b_fc.reshape(1, 1))
    return out


# speedup vs baseline: 14.5525x; 1.2742x over previous
import functools

import jax
import jax.numpy as jnp
from jax import lax
from jax.experimental import pallas as pl
from jax.experimental.pallas import tpu as pltpu
from jax.experimental.pallas import tpu_sc as plsc

N_NODES = 10000
N_EDGES = 640000
IN_DIM = 64
D = 2 * IN_DIM
HID = 128
N_GRAPHS = 16

NC = 2
NS = 16
NW = NC * NS

E_PER_SUB = N_EDGES // NW
CHUNK = 128
N_MAIN = E_PER_SUB // CHUNK
TAIL = E_PER_SUB - N_MAIN * CHUNK
UNROLL = 4

ROWS_PER_SUB = 624
ROWS_TAIL = N_NODES - ROWS_PER_SUB * NS
TAIL_BASE = ROWS_PER_SUB * NS

DEG_W = 16
N_PAD = 10240
DEG_ROWS = N_PAD // NS

_MESH = plsc.VectorSubcoreMesh(core_axis_name="c", subcore_axis_name="s")


def _sc_feature_sums(x, idx_main, idx_tail, zrows):
    @functools.partial(
        pl.kernel,
        out_type=jax.ShapeDtypeStruct((NC, N_NODES, D), jnp.float32),
        mesh=_MESH,
        scratch_types=[
            pltpu.VMEM((2, CHUNK), jnp.int32),
            pltpu.VMEM((2, CHUNK), jnp.int32),
            pltpu.VMEM((2, CHUNK), jnp.int32),
            pltpu.VMEM((2, CHUNK), jnp.int32),
            pltpu.VMEM((CHUNK, D), jnp.float32),
            pltpu.VMEM((CHUNK, D), jnp.float32),
            pltpu.VMEM((2, TAIL), jnp.int32),
            pltpu.VMEM((TAIL, D), jnp.float32),
            pltpu.VMEM_SHARED((N_NODES, D), jnp.float32),
            pltpu.SemaphoreType.DMA,
            pltpu.SemaphoreType.DMA,
            pltpu.SemaphoreType.DMA,
            pltpu.SemaphoreType.DMA,
            pltpu.SemaphoreType.DMA,
            pltpu.SemaphoreType.DMA,
            pltpu.SemaphoreType.DMA,
            pltpu.SemaphoreType.DMA,
        ],
    )
    def k(x_hbm, im_hbm, it_hbm, zr_hbm, acc_out,
          ibuf0, ibuf1, ibuf2, ibuf3, rows0, rows1, tbuf, trows, acc_sh,
          si0, si1, si2, si3, sg0, sg1, ss0, ss1):
        cid = lax.axis_index("c")
        sid = lax.axis_index("s")
        wid = cid * NS + sid
        base = sid * ROWS_PER_SUB
        ibuf = (ibuf0, ibuf1, ibuf2, ibuf3)
        rows = (rows0, rows1)
        si = (si0, si1, si2, si3)
        sg = (sg0, sg1)
        ss = (ss0, ss1)

        pltpu.sync_copy(zr_hbm.at[pl.ds(base, ROWS_PER_SUB)],
                        acc_sh.at[pl.ds(base, ROWS_PER_SUB)])

        @pl.when(sid == NS - 1)
        def _():
            pltpu.sync_copy(zr_hbm.at[pl.ds(TAIL_BASE, ROWS_TAIL)],
                            acc_sh.at[pl.ds(TAIL_BASE, ROWS_TAIL)])

        plsc.subcore_barrier()

        def idx_start(k_, t):
            pltpu.async_copy(im_hbm.at[wid, k_], ibuf[t], si[t])

        def idx_wait(t):
            pltpu.make_async_copy(im_hbm.at[wid, 0], ibuf[t], si[t]).wait()

        def gather_start(t, r):
            pltpu.async_copy(x_hbm.at[ibuf[t].at[0]], rows[r], sg[r])

        def gather_wait(t, r):
            pltpu.make_async_copy(x_hbm.at[ibuf[t].at[0]], rows[r],
                                  sg[r]).wait()

        def scatter_start(t, r):
            pltpu.async_copy(rows[r], acc_sh.at[ibuf[t].at[1]], ss[r],
                             add=True)

        def scatter_wait(t, r):
            pltpu.make_async_copy(rows[r], acc_sh.at[ibuf[t].at[1]],
                                  ss[r]).wait()

        idx_start(0, 0)
        idx_start(1, 1)

        @pl.loop(0, N_MAIN // UNROLL)
        def _(jj):
            for t in range(UNROLL):
                m = jj * UNROLL + t
                r = t % 2

                @pl.when(m >= 2)
                def _():
                    scatter_wait((t + 2) % 4, r)

                @pl.when(m + 2 < N_MAIN)
                def _():
                    idx_start(m + 2, (t + 2) % 4)

                idx_wait(t)
                gather_start(t, r)
                gather_wait(t, r)
                scatter_start(t, r)

        scatter_wait(2, 0)
        scatter_wait(3, 1)
        pltpu.sync_copy(it_hbm.at[wid], tbuf)
        pltpu.async_copy(x_hbm.at[tbuf.at[0]], trows, sg0).wait()
        pltpu.sync_copy(trows, acc_sh.at[tbuf.at[1]], add=True)

        plsc.subcore_barrier()
        pltpu.sync_copy(acc_sh.at[pl.ds(base, ROWS_PER_SUB)],
                        acc_out.at[cid, pl.ds(base, ROWS_PER_SUB)])

        @pl.when(sid == NS - 1)
        def _():
            pltpu.sync_copy(acc_sh.at[pl.ds(TAIL_BASE, ROWS_TAIL)],
                            acc_out.at[cid, pl.ds(TAIL_BASE, ROWS_TAIL)])

    return k(x, idx_main, idx_tail, zrows)


def _sc_degrees(idx_main, idx_tail, ones_chunk, zdeg):
    @functools.partial(
        pl.kernel,
        out_type=jax.ShapeDtypeStruct((NC, N_PAD, DEG_W), jnp.float32),
        mesh=_MESH,
        scratch_types=[
            pltpu.VMEM((2, CHUNK), jnp.int32),
            pltpu.VMEM((2, CHUNK), jnp.int32),
            pltpu.VMEM((2, CHUNK), jnp.int32),
            pltpu.VMEM((2, CHUNK), jnp.int32),
            pltpu.VMEM((CHUNK, DEG_W), jnp.float32),
            pltpu.VMEM((2, TAIL), jnp.int32),
            pltpu.VMEM_SHARED((N_PAD, DEG_W), jnp.float32),
            pltpu.SemaphoreType.DMA,
            pltpu.SemaphoreType.DMA,
            pltpu.SemaphoreType.DMA,
            pltpu.SemaphoreType.DMA,
            pltpu.SemaphoreType.DMA,
            pltpu.SemaphoreType.DMA,
        ],
        compiler_params=pltpu.CompilerParams(use_tc_tiling_on_sc=False),
    )
    def k(im_hbm, it_hbm, ones_hbm, zd_hbm, deg_out,
          ibuf0, ibuf1, ibuf2, ibuf3, ones_v, tbuf, deg_sh,
          si0, si1, si2, si3, ss0, ss1):
        cid = lax.axis_index("c")
        sid = lax.axis_index("s")
        wid = cid * NS + sid
        base = sid * DEG_ROWS
        ibuf = (ibuf0, ibuf1, ibuf2, ibuf3)
        si = (si0, si1, si2, si3)
        ss = (ss0, ss1)

        pltpu.sync_copy(zd_hbm.at[pl.ds(base, DEG_ROWS)],
                        deg_sh.at[pl.ds(base, DEG_ROWS)])
        pltpu.sync_copy(ones_hbm, ones_v)
        plsc.subcore_barrier()

        def idx_start(k_, t):
            pltpu.async_copy(im_hbm.at[wid, k_], ibuf[t], si[t])

        def idx_wait(t):
            pltpu.make_async_copy(im_hbm.at[wid, 0], ibuf[t], si[t]).wait()

        def scatter_start(t, r):
            pltpu.async_copy(ones_v, deg_sh.at[ibuf[t].at[1]], ss[r],
                             add=True)

        def scatter_wait(t, r):
            pltpu.make_async_copy(ones_v, deg_sh.at[ibuf[t].at[1]],
                                  ss[r]).wait()

        idx_start(0, 0)
        idx_start(1, 1)

        @pl.loop(0, N_MAIN // UNROLL)
        def _(jj):
            for t in range(UNROLL):
                m = jj * UNROLL + t
                r = t % 2

                @pl.when(m >= 2)
                def _():
                    scatter_wait((t + 2) % 4, r)

                @pl.when(m + 2 < N_MAIN)
                def _():
                    idx_start(m + 2, (t + 2) % 4)

                idx_wait(t)
                scatter_start(t, r)

        scatter_wait(2, 0)
        scatter_wait(3, 1)
        pltpu.sync_copy(it_hbm.at[wid], tbuf)
        pltpu.sync_copy(ones_v.at[pl.ds(0, TAIL)], deg_sh.at[tbuf.at[1]],
                        add=True)

        plsc.subcore_barrier()
        pltpu.sync_copy(deg_sh.at[pl.ds(base, DEG_ROWS)],
                        deg_out.at[cid, pl.ds(base, DEG_ROWS)])

    return k(idx_main, idx_tail, ones_chunk, zdeg)


def _tc_body(x_ref, acc_ref, deg_ref, batch_ref,
             wl_ref, bl_ref, wr_ref, wf1_ref, bf1_ref, wf_ref, bf_ref,
             out_ref):
    x = x_ref[...]
    acc = acc_ref[0] + acc_ref[1]
    deg = (deg_ref[0, 0:N_NODES, 0:1]
           + deg_ref[1, 0:N_NODES, 0:1])
    agg_mean = acc / jnp.maximum(deg, 1.0)
    x_gnn = jnp.maximum(
        jnp.dot(agg_mean, wl_ref[...], preferred_element_type=jnp.float32)
        + bl_ref[...]
        + jnp.dot(x, wr_ref[...], preferred_element_type=jnp.float32),
        0.0)
    x_mlp = jnp.maximum(
        jnp.dot(x, wf1_ref[...], preferred_element_type=jnp.float32)
        + bf1_ref[...],
        0.0)
    x_comb = x_gnn + x_mlp

    b = batch_ref[...]
    gids = lax.broadcasted_iota(jnp.int32, (1, N_GRAPHS), 1)
    onehot = (b == gids).astype(jnp.float32)
    g_sum = lax.dot_general(onehot, x_comb,
                            (((0,), (0,)), ((), ())),
                            preferred_element_type=jnp.float32)
    g_cnt = jnp.sum(onehot, axis=0)[:, None]
    gf = g_sum / jnp.maximum(g_cnt, 1.0)
    logits = jnp.dot(gf, wf_ref[...],
                     preferred_element_type=jnp.float32) + bf_ref[...]
    out_ref[...] = jax.nn.sigmoid(logits)


def kernel(normal_features, extreme_features, edge_index, batch,
           W_l, b_l, W_r, W_fc1, b_fc1, W_fc, b_fc):
    x = jnp.concatenate([normal_features, extreme_features], axis=1)
    src = edge_index[0].astype(jnp.int32).reshape(NW, E_PER_SUB)
    dst = edge_index[1].astype(jnp.int32).reshape(NW, E_PER_SUB)
    idx_main = jnp.stack(
        [src[:, :N_MAIN * CHUNK].reshape(NW, N_MAIN, CHUNK),
         dst[:, :N_MAIN * CHUNK].reshape(NW, N_MAIN, CHUNK)], axis=2)
    idx_tail = jnp.stack(
        [src[:, N_MAIN * CHUNK:], dst[:, N_MAIN * CHUNK:]], axis=1)
    zrows = jnp.zeros((N_NODES, D), jnp.float32)
    zdeg = jnp.zeros((N_PAD, DEG_W), jnp.float32)
    ones_chunk = jnp.ones((CHUNK, DEG_W), jnp.float32)

    acc2 = _sc_feature_sums(x, idx_main, idx_tail, zrows)
    deg2 = _sc_degrees(idx_main, idx_tail, ones_chunk, zdeg)

    out = pl.pallas_call(
        _tc_body,
        out_shape=jax.ShapeDtypeStruct((N_GRAPHS, 1), jnp.float32),
    )(x, acc2, deg2, batch.astype(jnp.int32).reshape(N_NODES, 1),
      W_l, b_l.reshape(1, HID), W_r, W_fc1, b_fc1.reshape(1, HID),
      W_fc, b_fc.reshape(1, 1))
    return out


# speedup vs baseline: 15.0266x; 1.0326x over previous
import functools

import jax
import jax.numpy as jnp
from jax import lax
from jax.experimental import pallas as pl
from jax.experimental.pallas import tpu as pltpu
from jax.experimental.pallas import tpu_sc as plsc

N_NODES = 10000
N_EDGES = 640000
IN_DIM = 64
D = 2 * IN_DIM
HID = 128
N_GRAPHS = 16

NC = 2
NS = 16
NW = NC * NS

E_PER_SUB = N_EDGES // NW
CHUNK = 128
N_MAIN = E_PER_SUB // CHUNK
TAIL = E_PER_SUB - N_MAIN * CHUNK
UNROLL = 4

ROWS_PER_SUB = 624
ROWS_TAIL = N_NODES - ROWS_PER_SUB * NS
TAIL_BASE = ROWS_PER_SUB * NS

DEG_W = 16
N_PAD = 10240
DEG_ROWS = N_PAD // NS

_MESH = plsc.VectorSubcoreMesh(core_axis_name="c", subcore_axis_name="s")


def _sc_feature_sums(x, idx_main, idx_tail, zrows, zdeg, ones_chunk):
    @functools.partial(
        pl.kernel,
        out_type=(
            jax.ShapeDtypeStruct((NC, N_NODES, D), jnp.float32),
            jax.ShapeDtypeStruct((NC, N_PAD, DEG_W), jnp.float32),
        ),
        mesh=_MESH,
        scratch_types=[
            pltpu.VMEM((2, CHUNK), jnp.int32),
            pltpu.VMEM((2, CHUNK), jnp.int32),
            pltpu.VMEM((2, CHUNK), jnp.int32),
            pltpu.VMEM((2, CHUNK), jnp.int32),
            pltpu.VMEM((CHUNK, D), jnp.float32),
            pltpu.VMEM((CHUNK, D), jnp.float32),
            pltpu.VMEM((2, TAIL), jnp.int32),
            pltpu.VMEM((TAIL, D), jnp.float32),
            pltpu.VMEM((CHUNK, DEG_W), jnp.float32),
            pltpu.VMEM_SHARED((N_NODES, D), jnp.float32),
            pltpu.VMEM_SHARED((N_PAD, DEG_W), jnp.float32),
            pltpu.SemaphoreType.DMA,
            pltpu.SemaphoreType.DMA,
            pltpu.SemaphoreType.DMA,
            pltpu.SemaphoreType.DMA,
            pltpu.SemaphoreType.DMA,
            pltpu.SemaphoreType.DMA,
            pltpu.SemaphoreType.DMA,
            pltpu.SemaphoreType.DMA,
            pltpu.SemaphoreType.DMA,
            pltpu.SemaphoreType.DMA,
        ],
        compiler_params=pltpu.CompilerParams(use_tc_tiling_on_sc=False),
    )
    def k(x_hbm, im_hbm, it_hbm, zr_hbm, zd_hbm, ones_hbm, acc_out, deg_out,
          ibuf0, ibuf1, ibuf2, ibuf3, rows0, rows1, tbuf, trows, ones_v,
          acc_sh, deg_sh,
          si0, si1, si2, si3, sg0, sg1, ss0, ss1, sd0, sd1):
        cid = lax.axis_index("c")
        sid = lax.axis_index("s")
        wid = cid * NS + sid
        base = sid * ROWS_PER_SUB
        dbase = sid * DEG_ROWS
        ibuf = (ibuf0, ibuf1, ibuf2, ibuf3)
        rows = (rows0, rows1)
        si = (si0, si1, si2, si3)
        sg = (sg0, sg1)
        ss = (ss0, ss1)
        sd = (sd0, sd1)

        pltpu.sync_copy(zr_hbm.at[pl.ds(base, ROWS_PER_SUB)],
                        acc_sh.at[pl.ds(base, ROWS_PER_SUB)])

        @pl.when(sid == NS - 1)
        def _():
            pltpu.sync_copy(zr_hbm.at[pl.ds(TAIL_BASE, ROWS_TAIL)],
                            acc_sh.at[pl.ds(TAIL_BASE, ROWS_TAIL)])

        pltpu.sync_copy(zd_hbm.at[pl.ds(dbase, DEG_ROWS)],
                        deg_sh.at[pl.ds(dbase, DEG_ROWS)])
        pltpu.sync_copy(ones_hbm, ones_v)
        plsc.subcore_barrier()

        def idx_start(k_, t):
            pltpu.async_copy(im_hbm.at[wid, k_], ibuf[t], si[t])

        def idx_wait(t):
            pltpu.make_async_copy(im_hbm.at[wid, 0], ibuf[t], si[t]).wait()

        def gather_start(t, r):
            pltpu.async_copy(x_hbm.at[ibuf[t].at[0]], rows[r], sg[r])

        def gather_wait(t, r):
            pltpu.make_async_copy(x_hbm.at[ibuf[t].at[0]], rows[r],
                                  sg[r]).wait()

        def scatter_start(t, r):
            pltpu.async_copy(rows[r], acc_sh.at[ibuf[t].at[1]], ss[r],
                             add=True)
            pltpu.async_copy(ones_v, deg_sh.at[ibuf[t].at[1]], sd[r],
                             add=True)

        def scatter_wait(t, r):
            pltpu.make_async_copy(rows[r], acc_sh.at[ibuf[t].at[1]],
                                  ss[r]).wait()
            pltpu.make_async_copy(ones_v, deg_sh.at[ibuf[t].at[1]],
                                  sd[r]).wait()

        idx_start(0, 0)
        idx_start(1, 1)

        @pl.loop(0, N_MAIN // UNROLL)
        def _(jj):
            for t in range(UNROLL):
                m = jj * UNROLL + t
                r = t % 2

                @pl.when(m >= 2)
                def _():
                    scatter_wait((t + 2) % 4, r)

                @pl.when(m + 2 < N_MAIN)
                def _():
                    idx_start(m + 2, (t + 2) % 4)

                idx_wait(t)
                gather_start(t, r)
                gather_wait(t, r)
                scatter_start(t, r)

        scatter_wait(2, 0)
        scatter_wait(3, 1)
        pltpu.sync_copy(it_hbm.at[wid], tbuf)
        pltpu.async_copy(x_hbm.at[tbuf.at[0]], trows, sg0).wait()
        pltpu.sync_copy(trows, acc_sh.at[tbuf.at[1]], add=True)
        pltpu.sync_copy(ones_v.at[pl.ds(0, TAIL)], deg_sh.at[tbuf.at[1]],
                        add=True)

        plsc.subcore_barrier()
        pltpu.sync_copy(acc_sh.at[pl.ds(base, ROWS_PER_SUB)],
                        acc_out.at[cid, pl.ds(base, ROWS_PER_SUB)])

        @pl.when(sid == NS - 1)
        def _():
            pltpu.sync_copy(acc_sh.at[pl.ds(TAIL_BASE, ROWS_TAIL)],
                            acc_out.at[cid, pl.ds(TAIL_BASE, ROWS_TAIL)])

        pltpu.sync_copy(deg_sh.at[pl.ds(dbase, DEG_ROWS)],
                        deg_out.at[cid, pl.ds(dbase, DEG_ROWS)])

    return k(x, idx_main, idx_tail, zrows, zdeg, ones_chunk)


def _tc_body(x_ref, acc_ref, deg_ref, batch_ref,
             wl_ref, bl_ref, wr_ref, wf1_ref, bf1_ref, wf_ref, bf_ref,
             out_ref):
    x = x_ref[...]
    acc = acc_ref[0] + acc_ref[1]
    deg = (deg_ref[0, 0:N_NODES, 0:1]
           + deg_ref[1, 0:N_NODES, 0:1])
    agg_mean = acc / jnp.maximum(deg, 1.0)
    x_gnn = jnp.maximum(
        jnp.dot(agg_mean, wl_ref[...], preferred_element_type=jnp.float32)
        + bl_ref[...]
        + jnp.dot(x, wr_ref[...], preferred_element_type=jnp.float32),
        0.0)
    x_mlp = jnp.maximum(
        jnp.dot(x, wf1_ref[...], preferred_element_type=jnp.float32)
        + bf1_ref[...],
        0.0)
    x_comb = x_gnn + x_mlp

    b = batch_ref[...]
    gids = lax.broadcasted_iota(jnp.int32, (1, N_GRAPHS), 1)
    onehot = (b == gids).astype(jnp.float32)
    g_sum = lax.dot_general(onehot, x_comb,
                            (((0,), (0,)), ((), ())),
                            preferred_element_type=jnp.float32)
    g_cnt = jnp.sum(onehot, axis=0)[:, None]
    gf = g_sum / jnp.maximum(g_cnt, 1.0)
    logits = jnp.dot(gf, wf_ref[...],
                     preferred_element_type=jnp.float32) + bf_ref[...]
    out_ref[...] = jax.nn.sigmoid(logits)


def kernel(normal_features, extreme_features, edge_index, batch,
           W_l, b_l, W_r, W_fc1, b_fc1, W_fc, b_fc):
    x = jnp.concatenate([normal_features, extreme_features], axis=1)
    src = edge_index[0].astype(jnp.int32).reshape(NW, E_PER_SUB)
    dst = edge_index[1].astype(jnp.int32).reshape(NW, E_PER_SUB)
    idx_main = jnp.stack(
        [src[:, :N_MAIN * CHUNK].reshape(NW, N_MAIN, CHUNK),
         dst[:, :N_MAIN * CHUNK].reshape(NW, N_MAIN, CHUNK)], axis=2)
    idx_tail = jnp.stack(
        [src[:, N_MAIN * CHUNK:], dst[:, N_MAIN * CHUNK:]], axis=1)
    zrows = jnp.zeros((N_NODES, D), jnp.float32)
    zdeg = jnp.zeros((N_PAD, DEG_W), jnp.float32)
    ones_chunk = jnp.ones((CHUNK, DEG_W), jnp.float32)

    acc2, deg2 = _sc_feature_sums(x, idx_main, idx_tail, zrows, zdeg,
                                  ones_chunk)

    out = pl.pallas_call(
        _tc_body,
        out_shape=jax.ShapeDtypeStruct((N_GRAPHS, 1), jnp.float32),
    )(x, acc2, deg2, batch.astype(jnp.int32).reshape(N_NODES, 1),
      W_l, b_l.reshape(1, HID), W_r, W_fc1, b_fc1.reshape(1, HID),
      W_fc, b_fc.reshape(1, 1))
    return out
